# Initial kernel scaffold; baseline (speedup 1.0000x reference)
import jax
import jax.numpy as jnp
from jax.experimental import pallas as pl


def kernel(pred_tensor, target_tensor):
    raise NotImplementedError("write your pallas kernel here")


# speedup vs baseline: 8.6392x; 8.6392x over previous
import functools

import jax
import jax.numpy as jnp
from jax import lax
from jax.experimental import pallas as pl
from jax.experimental.pallas import tpu as pltpu
from jax.experimental.pallas import tpu_sc as plsc

NC = 2
NS = 16
NW = NC * NS
LANES = 16

ROW_F = 26
M_ROWS = 1024 * 7 * 7
ROWS_PER = M_ROWS // NW
ELEMS_PER = ROWS_PER * ROW_F
GROUPS = ROWS_PER // LANES
LIST_PAD = ROWS_PER + 2 * LANES

GRID_LEN = 448.0 / 7.0
HALF_IMG = 0.5 * 448.0


def _sqrt_nr(s):
    bits = lax.bitcast_convert_type(s, jnp.int32)
    y = lax.bitcast_convert_type(jnp.int32(0x5F3759DF) - (bits >> 1), jnp.float32)
    for _ in range(3):
        y = y * (1.5 - 0.5 * s * y * y)
    return s * y


def _box_edges(x, y, w, h):
    xc = x * GRID_LEN
    yc = y * GRID_LEN
    wh = w * HALF_IMG
    hh = h * HALF_IMG
    return xc - wh, xc + wh, yc - hh, yc + hh


def _stage1_body(pred_hbm, tgt_hbm, summ_hbm, pred_v, tgt_v, c_list, g_list, g_shift, summ_v):
    cid = lax.axis_index("c")
    sid = lax.axis_index("s")
    wid = sid * NC + cid
    base = wid * ELEMS_PER
    pltpu.sync_copy(pred_hbm.at[pl.ds(base, ELEMS_PER)], pred_v)
    pltpu.sync_copy(tgt_hbm.at[pl.ds(base, ELEMS_PER)], tgt_v)

    iota = lax.iota(jnp.int32, LANES)

    def group(j, carry):
        ptr, acc = carry
        off = iota * ROW_F + j * (LANES * ROW_F)

        def gp(f):
            return plsc.load_gather(pred_v, [off + f])

        def gt(f):
            return plsc.load_gather(tgt_v, [off + f])

        p0x, p0y, p0w, p0h, p0c = gp(0), gp(1), gp(2), gp(3), gp(4)
        p1x, p1y, p1w, p1h, p1c = gp(5), gp(6), gp(7), gp(8), gp(9)
        t0x, t0y, t0w, t0h, t0c = gt(0), gt(1), gt(2), gt(3), gt(4)
        t1x, t1y, t1w, t1h, t1c = gt(5), gt(6), gt(7), gt(8), gt(9)

        def iou(px, py, pw, ph, tx, ty, tw, th):
            pxn, pxx, pyn, pyx = _box_edges(px, py, pw, ph)
            txn, txx, tyn, tyx = _box_edges(tx, ty, tw, th)
            iw = jnp.maximum(jnp.minimum(pxx, txx) - jnp.maximum(pxn, txn), 0.0)
            ih = jnp.maximum(jnp.minimum(pyx, tyx) - jnp.maximum(pyn, tyn), 0.0)
            ai = iw * ih
            a1 = jnp.maximum(pxx - pxn, 0.0) * jnp.maximum(pyx - pyn, 0.0)
            a2 = jnp.maximum(txx - txn, 0.0) * jnp.maximum(tyx - tyn, 0.0)
            return ai / (a1 + a2 - ai + 1e-6)

        iou0 = iou(p0x, p0y, p0w, p0h, t0x, t0y, t0w, t0h)
        iou1 = iou(p1x, p1y, p1w, p1h, t1x, t1y, t1w, t1h)

        sel = iou1 > iou0
        miou = jnp.maximum(iou0, iou1)
        rpx = jnp.where(sel, p1x, p0x)
        rpy = jnp.where(sel, p1y, p0y)
        rpw = jnp.where(sel, p1w, p0w)
        rph = jnp.where(sel, p1h, p0h)
        rpc = jnp.where(sel, p1c, p0c)
        rtx = jnp.where(sel, t1x, t0x)
        rty = jnp.where(sel, t1y, t0y)
        rtw = jnp.where(sel, t1w, t0w)
        rth = jnp.where(sel, t1h, t0h)

        conf = t0c
        activeb = conf > 0.0
        cm = jnp.where(activeb, 1.0, 0.0)

        dx = rpx - rtx
        dy = rpy - rty
        loc = dx * dx + dy * dy
        loc = loc + rpw + rtw - 2.0 * _sqrt_nr(rpw * rtw)
        loc = loc + rph + rth - 2.0 * _sqrt_nr(rph * rth)

        dc = rpc - miou
        confl = dc * dc
        own = jnp.where(sel, p0c * p0c, 0.0)
        gterm = jnp.where(sel, 0.0, 1.0)
        cvec = p1c * p1c

        dno = p0c - t0c
        noobj = dno * dno

        cls_sum = jnp.zeros((LANES,), jnp.float32)
        for k in range(16):
            dk = gp(10 + k) - gt(10 + k)
            cls_sum = cls_sum + dk * dk

        acc = acc + cm * (5.0 * loc + confl + own + cls_sum) \
                  + 0.5 * ((1.0 - cm) * noobj)

        pos = plsc.cumsum(cm).astype(jnp.int32)
        idxc = pos - 1 + ptr
        plsc.store_scatter(c_list, [idxc], cvec, mask=activeb)
        plsc.store_scatter(g_list, [idxc], gterm, mask=activeb)
        idxs = idxc - 1
        plsc.store_scatter(g_shift, [idxs], gterm, mask=activeb & (idxs >= 0))
        ptr = ptr + jnp.max(pos)
        return ptr, acc

    ptr, acc = lax.fori_loop(
        0, GROUPS, group, (jnp.int32(0), jnp.zeros((LANES,), jnp.float32)))
    partial = jnp.sum(acc)

    K = ptr
    nblk = (K + 14) >> 4

    def pairblk(b, pacc):
        cv = c_list[pl.ds(b * LANES, LANES)]
        gv = g_shift[pl.ds(b * LANES, LANES)]
        jv = b * LANES + iota
        return pacc + jnp.where(jv < K - 1, cv * gv, 0.0)

    pacc = lax.fori_loop(0, nblk, pairblk, jnp.zeros((LANES,), jnp.float32))
    partial = partial + jnp.sum(pacc)

    has = jnp.where(K > 0, 1.0, 0.0)
    first_g = g_list[pl.ds(0, LANES)][0]
    last_c = c_list[pl.ds(jnp.maximum(K - 1, 0), LANES)][0]
    summ_v[...] = jnp.where(
        iota == 0, partial,
        jnp.where(iota == 1, has,
                  jnp.where(iota == 2, first_g,
                            jnp.where(iota == 3, last_c, 0.0))))
    pltpu.sync_copy(summ_v, summ_hbm.at[wid])


def _stage2_body(summ_hbm, out_hbm, all_v, out_v):
    cid = lax.axis_index("c")
    sid = lax.axis_index("s")
    wid = sid * NC + cid

    @pl.when(wid == 0)
    def _():
        pltpu.sync_copy(summ_hbm, all_v)

        def link(w, carry):
            total, seen, firstg, lastc = carry
            row = all_v[w]
            p = row[0]
            has = row[1]
            fg = row[2]
            lc = row[3]
            total = total + p + has * seen * lastc * fg
            firstg = firstg + (1.0 - seen) * has * fg
            lastc = lastc + has * (lc - lastc)
            seen = jnp.maximum(seen, has)
            return total, seen, firstg, lastc

        total, seen, firstg, lastc = lax.fori_loop(
            0, NW, link,
            (jnp.float32(0.0), jnp.float32(0.0), jnp.float32(0.0), jnp.float32(0.0)))
        total = (total + seen * lastc * firstg) * (1.0 / 1024.0)
        iota = lax.iota(jnp.int32, LANES)
        out_v[...] = jnp.where(iota == 0, total, 0.0)
        pltpu.sync_copy(out_v, out_hbm)


_stage1 = functools.partial(
    pl.kernel,
    out_type=jax.ShapeDtypeStruct((NW, LANES), jnp.float32),
    mesh=plsc.VectorSubcoreMesh(core_axis_name="c", subcore_axis_name="s"),
    scratch_types=[
        pltpu.VMEM((ELEMS_PER,), jnp.float32),
        pltpu.VMEM((ELEMS_PER,), jnp.float32),
        pltpu.VMEM((LIST_PAD,), jnp.float32),
        pltpu.VMEM((LIST_PAD,), jnp.float32),
        pltpu.VMEM((LIST_PAD,), jnp.float32),
        pltpu.VMEM((LANES,), jnp.float32),
    ],
    compiler_params=pltpu.CompilerParams(needs_layout_passes=False),
)(_stage1_body)

_stage2 = functools.partial(
    pl.kernel,
    out_type=jax.ShapeDtypeStruct((LANES,), jnp.float32),
    mesh=plsc.VectorSubcoreMesh(core_axis_name="c", subcore_axis_name="s"),
    scratch_types=[
        pltpu.VMEM((NW, LANES), jnp.float32),
        pltpu.VMEM((LANES,), jnp.float32),
    ],
    compiler_params=pltpu.CompilerParams(needs_layout_passes=False),
)(_stage2_body)


def kernel(pred_tensor, target_tensor):
    fp = pred_tensor.reshape(-1)
    ft = target_tensor.reshape(-1)
    summ = _stage1(fp, ft)
    out = _stage2(summ)
    return out[0]
